# baseline (speedup 1.0000x reference)
import functools
import math

import jax
import jax.numpy as jnp
from jax import lax
from jax.experimental import pallas as pl
from jax.experimental.pallas import tpu as pltpu
from jax.experimental.pallas import tpu_sc as plsc

D_MODEL = 1024
LANES = 16
SCALE = math.sqrt(D_MODEL)
NBUF = 4
AHEAD = 2


@functools.partial(jax.jit, static_argnums=(2, 3, 4))
def _sc_embed(tok, table, B, NC, NS):
    NW = NC * NS
    tok_cols = tok.shape[-1]
    rows_per_w = B // NW
    C = 16
    nchunk = rows_per_w // C
    groups = D_MODEL // LANES

    mesh = plsc.VectorSubcoreMesh(core_axis_name="c", subcore_axis_name="s")

    @functools.partial(
        pl.kernel,
        out_type=jax.ShapeDtypeStruct((B, D_MODEL), jnp.float32),
        mesh=mesh,
        scratch_types=[
            pltpu.VMEM((rows_per_w,), jnp.int32),
            pltpu.VMEM((NBUF, C, D_MODEL), jnp.float32),
        ] + [pltpu.SemaphoreType.DMA] * (2 * NBUF),
    )
    def emb_kernel(tok_hbm, table_hbm, out_hbm, idx_v, bufs, *sems):
        gsem = sems[:NBUF]
        ssem = sems[NBUF:]
        wid = lax.axis_index("s") * NC + lax.axis_index("c")
        base = wid * rows_per_w
        w_per_row = tok_cols // rows_per_w
        pltpu.sync_copy(
            tok_hbm.at[wid // w_per_row,
                       pl.ds((wid % w_per_row) * rows_per_w, rows_per_w)],
            idx_v)

        def gather(j, b, sem):
            return pltpu.async_copy(
                table_hbm.at[idx_v.at[pl.ds(j * C, C)]], bufs.at[b], sem)

        def store_desc(j, b, sem):
            return pltpu.make_async_copy(
                bufs.at[b], out_hbm.at[pl.ds(base + j * C, C)], sem)

        for p in range(AHEAD):
            gather(p, p, gsem[p])

        def step(j, b):
            b2 = (b + AHEAD) % NBUF

            @pl.when(j >= NBUF - AHEAD)
            def _():
                store_desc(j - (NBUF - AHEAD), b2, ssem[b2]).wait()

            @pl.when(j + AHEAD < nchunk)
            def _():
                gather(j + AHEAD, b2, gsem[b2])

            pltpu.make_async_copy(
                table_hbm.at[idx_v.at[pl.ds(j * C, C)]],
                bufs.at[b], gsem[b]).wait()

            def row_body(r, c2):
                for q in range(groups):
                    sl = pl.ds(q * LANES, LANES)
                    bufs[b, r, sl] = bufs[b, r, sl] * SCALE
                return c2

            lax.fori_loop(0, C, row_body, 0, unroll=False)
            store_desc(j, b, ssem[b]).start()

        def quad(j4, carry):
            for b in range(NBUF):
                step(j4 * NBUF + b, b)
            return carry

        lax.fori_loop(0, nchunk // NBUF, quad, 0, unroll=False)

        for j in range(nchunk - (NBUF - AHEAD), nchunk):
            store_desc(j, j % NBUF, ssem[j % NBUF]).wait()

    return emb_kernel(tok, table)


def kernel(tokens, embedding):
    B = tokens.size
    try:
        info = plsc.get_sparse_core_info()
        NC, NS = info.num_cores, info.num_subcores
    except Exception:
        NC, NS = 2, 16
    out = _sc_embed(tokens.astype(jnp.int32), embedding, B, NC, NS)
    return out.reshape(tokens.shape + (D_MODEL,))


# speedup vs baseline: 1.0133x; 1.0133x over previous
import functools
import math

import jax
import jax.numpy as jnp
from jax import lax
from jax.experimental import pallas as pl
from jax.experimental.pallas import tpu as pltpu
from jax.experimental.pallas import tpu_sc as plsc

D_MODEL = 1024
LANES = 16
SCALE = math.sqrt(D_MODEL)
NBUF = 8
AHEAD = 5


@functools.partial(jax.jit, static_argnums=(2, 3, 4))
def _sc_embed(tok, table, B, NC, NS):
    NW = NC * NS
    tok_cols = tok.shape[-1]
    rows_per_w = B // NW
    C = 8
    nchunk = rows_per_w // C
    groups = D_MODEL // LANES

    mesh = plsc.VectorSubcoreMesh(core_axis_name="c", subcore_axis_name="s")

    @functools.partial(
        pl.kernel,
        out_type=jax.ShapeDtypeStruct((B, D_MODEL), jnp.float32),
        mesh=mesh,
        scratch_types=[
            pltpu.VMEM((rows_per_w,), jnp.int32),
            pltpu.VMEM((NBUF, C, D_MODEL), jnp.float32),
        ] + [pltpu.SemaphoreType.DMA] * (2 * NBUF),
    )
    def emb_kernel(tok_hbm, table_hbm, out_hbm, idx_v, bufs, *sems):
        gsem = sems[:NBUF]
        ssem = sems[NBUF:]
        wid = lax.axis_index("s") * NC + lax.axis_index("c")
        base = wid * rows_per_w
        w_per_row = tok_cols // rows_per_w
        pltpu.sync_copy(
            tok_hbm.at[wid // w_per_row,
                       pl.ds((wid % w_per_row) * rows_per_w, rows_per_w)],
            idx_v)

        def gather(j, b, sem):
            return pltpu.async_copy(
                table_hbm.at[idx_v.at[pl.ds(j * C, C)]], bufs.at[b], sem)

        def store_desc(j, b, sem):
            return pltpu.make_async_copy(
                bufs.at[b], out_hbm.at[pl.ds(base + j * C, C)], sem)

        for p in range(AHEAD):
            gather(p, p, gsem[p])

        def step(j, b):
            b2 = (b + AHEAD) % NBUF

            @pl.when(j >= NBUF - AHEAD)
            def _():
                store_desc(j - (NBUF - AHEAD), b2, ssem[b2]).wait()

            @pl.when(j + AHEAD < nchunk)
            def _():
                gather(j + AHEAD, b2, gsem[b2])

            pltpu.make_async_copy(
                table_hbm.at[idx_v.at[pl.ds(j * C, C)]],
                bufs.at[b], gsem[b]).wait()

            def row_body(r, c2):
                for q in range(groups):
                    sl = pl.ds(q * LANES, LANES)
                    bufs[b, r, sl] = bufs[b, r, sl] * SCALE
                return c2

            lax.fori_loop(0, C, row_body, 0, unroll=False)
            store_desc(j, b, ssem[b]).start()

        def quad(j4, carry):
            for b in range(NBUF):
                step(j4 * NBUF + b, b)
            return carry

        lax.fori_loop(0, nchunk // NBUF, quad, 0, unroll=False)

        for j in range(nchunk - (NBUF - AHEAD), nchunk):
            store_desc(j, j % NBUF, ssem[j % NBUF]).wait()

    return emb_kernel(tok, table)


def kernel(tokens, embedding):
    B = tokens.size
    try:
        info = plsc.get_sparse_core_info()
        NC, NS = info.num_cores, info.num_subcores
    except Exception:
        NC, NS = 2, 16
    out = _sc_embed(tokens.astype(jnp.int32), embedding, B, NC, NS)
    return out.reshape(tokens.shape + (D_MODEL,))
